# baseline (speedup 1.0000x reference)
import functools

import jax
import jax.numpy as jnp
from jax import lax
from jax.experimental import pallas as pl
from jax.experimental.pallas import tpu as pltpu
from jax.experimental.pallas import tpu_sc as plsc

D = 128
LANES = 16
CHUNK = 128


@functools.partial(jax.jit, static_argnums=())
def _embed_sum(ids, pids, sids, word_emb, pos_emb, sent_emb):
    N = ids.shape[0]
    info = plsc.get_sparse_core_info()
    NC, NS = info.num_cores, info.num_subcores
    NW = NC * NS
    per_w = N // NW
    n_chunks = per_w // CHUNK
    assert per_w * NW == N and n_chunks * CHUNK == per_w

    mesh = plsc.VectorSubcoreMesh(core_axis_name="c", subcore_axis_name="s")

    @functools.partial(
        pl.kernel,
        mesh=mesh,
        out_type=jax.ShapeDtypeStruct((N, D), jnp.float32),
        scratch_types=[
            pltpu.VMEM((CHUNK,), jnp.int32),
            pltpu.VMEM((CHUNK,), jnp.int32),
            pltpu.VMEM((CHUNK,), jnp.int32),
            pltpu.VMEM((CHUNK, D), jnp.float32),
            pltpu.VMEM((CHUNK, D), jnp.float32),
            pltpu.VMEM((CHUNK, D), jnp.float32),
            pltpu.SemaphoreType.DMA,
            pltpu.SemaphoreType.DMA,
            pltpu.SemaphoreType.DMA,
        ],
    )
    def k(ids_hbm, pids_hbm, sids_hbm, word_hbm, pos_hbm, sent_hbm, out_hbm,
          widx, pidx, sidx, wrows, prows, srows, sem_w, sem_p, sem_s):
        wid = lax.axis_index("s") * NC + lax.axis_index("c")
        base = wid * per_w

        def chunk_body(c, _):
            off = base + c * CHUNK
            pltpu.sync_copy(ids_hbm.at[pl.ds(off, CHUNK)], widx)
            pltpu.sync_copy(pids_hbm.at[pl.ds(off, CHUNK)], pidx)
            pltpu.sync_copy(sids_hbm.at[pl.ds(off, CHUNK)], sidx)
            cw = pltpu.async_copy(word_hbm.at[widx], wrows, sem_w)
            cp = pltpu.async_copy(pos_hbm.at[pidx], prows, sem_p)
            cs = pltpu.async_copy(sent_hbm.at[sidx], srows, sem_s)
            cw.wait()
            cp.wait()
            cs.wait()

            def add_body(r, _):
                for j in range(D // LANES):
                    sl = pl.ds(j * LANES, LANES)
                    wrows[r, sl] = wrows[r, sl] + prows[r, sl] + srows[r, sl]
                return 0

            lax.fori_loop(0, CHUNK, add_body, 0)
            pltpu.sync_copy(wrows, out_hbm.at[pl.ds(off, CHUNK)])
            return 0

        lax.fori_loop(0, n_chunks, chunk_body, 0)

    return k(ids, pids, sids, word_emb, pos_emb, sent_emb)


def kernel(input_ids, sent_ids_tensor, position_ids, word_embedding,
           pos_embedding, sent_embedding):
    B, S = input_ids.shape
    N = B * S
    ids = input_ids.reshape(N).astype(jnp.int32)
    pids = position_ids.reshape(N).astype(jnp.int32)
    sids = sent_ids_tensor.reshape(N).astype(jnp.int32)
    out = _embed_sum(ids, pids, sids, word_embedding, pos_embedding,
                     sent_embedding)
    return out.reshape(B, S, D)
